# Initial kernel scaffold; baseline (speedup 1.0000x reference)
import jax
import jax.numpy as jnp
from jax.experimental import pallas as pl


def kernel(p_sequences, table):
    raise NotImplementedError("write your pallas kernel here")


# speedup vs baseline: 4.9461x; 4.9461x over previous
import functools

import jax
import jax.numpy as jnp
from jax import lax
from jax.experimental import pallas as pl
from jax.experimental.pallas import tpu as pltpu
from jax.experimental.pallas import tpu_sc as plsc

EMB_DIM = 64


def _make_lookup(B_flat: int, V: int, D: int):
    info = plsc.get_sparse_core_info()
    NC, NS = info.num_cores, info.num_subcores
    NW = NC * NS
    assert B_flat % NW == 0
    b_per_w = B_flat // NW
    chunk = 1024
    while b_per_w % chunk:
        chunk //= 2
    n_chunks = b_per_w // chunk

    mesh = plsc.VectorSubcoreMesh(core_axis_name="c", subcore_axis_name="s")

    @functools.partial(
        pl.kernel,
        mesh=mesh,
        out_type=jax.ShapeDtypeStruct((B_flat, D), jnp.float32),
        scratch_types=[
            pltpu.VMEM((chunk,), jnp.int32),
            pltpu.VMEM((chunk, D), jnp.float32),
            pltpu.SemaphoreType.DMA,
        ],
        compiler_params=pltpu.CompilerParams(use_tc_tiling_on_sc=False),
    )
    def lookup(idx_hbm, table_hbm, out_hbm, idx_v, rows_v, sem):
        wid = lax.axis_index("s") * NC + lax.axis_index("c")
        base = wid * b_per_w

        def body(i, _):
            off = base + i * chunk
            pltpu.sync_copy(idx_hbm.at[pl.ds(off, chunk)], idx_v)
            pltpu.async_copy(table_hbm.at[idx_v], rows_v, sem).wait()
            pltpu.sync_copy(rows_v, out_hbm.at[pl.ds(off, chunk)])
            return ()

        lax.fori_loop(0, n_chunks, body, (), unroll=False)

    return lookup


def kernel(p_sequences, table):
    B, L = p_sequences.shape
    V, D = table.shape
    idx_flat = p_sequences.reshape(B * L)
    lookup = _make_lookup(B * L, V, D)
    out = lookup(idx_flat, table)
    return out.reshape(B, L, D)


# speedup vs baseline: 5.0939x; 1.0299x over previous
import functools

import jax
import jax.numpy as jnp
from jax import lax
from jax.experimental import pallas as pl
from jax.experimental.pallas import tpu as pltpu
from jax.experimental.pallas import tpu_sc as plsc

CHUNK = 512
NBUF = 2


def _make_lookup(B_flat: int, D: int):
    info = plsc.get_sparse_core_info()
    NC, NS = info.num_cores, info.num_subcores
    NW = NC * NS
    b_per_w = B_flat // NW
    group = CHUNK * NBUF
    assert B_flat % NW == 0 and b_per_w % group == 0
    n_groups = b_per_w // group

    mesh = plsc.VectorSubcoreMesh(core_axis_name="c", subcore_axis_name="s")

    @functools.partial(
        pl.kernel,
        mesh=mesh,
        out_type=jax.ShapeDtypeStruct((B_flat, D), jnp.float32),
        scratch_types=[
            pltpu.VMEM((group,), jnp.int32),
            [pltpu.VMEM((CHUNK, D), jnp.float32) for _ in range(NBUF)],
            [pltpu.SemaphoreType.DMA for _ in range(NBUF)],
            [pltpu.SemaphoreType.DMA for _ in range(NBUF)],
        ],
        compiler_params=pltpu.CompilerParams(use_tc_tiling_on_sc=False),
    )
    def lookup(idx_hbm, table_hbm, out_hbm, idx_v, rows, gsem, ssem):
        wid = lax.axis_index("s") * NC + lax.axis_index("c")
        base = wid * b_per_w

        def idx_slice(b):
            return idx_v.at[pl.ds(b * CHUNK, CHUNK)]

        def start_gather(b):
            pltpu.async_copy(table_hbm.at[idx_slice(b)], rows[b], gsem[b])

        def wait_gather(b):
            pltpu.make_async_copy(table_hbm.at[idx_slice(b)], rows[b],
                                  gsem[b]).wait()

        def out_ref(off, b):
            return out_hbm.at[pl.ds(off + b * CHUNK, CHUNK)]

        def start_store(off, b):
            pltpu.async_copy(rows[b], out_ref(off, b), ssem[b])

        def wait_store(off, b):
            pltpu.make_async_copy(rows[b], out_ref(off, b), ssem[b]).wait()

        pltpu.sync_copy(idx_hbm.at[pl.ds(base, group)], idx_v)
        for b in range(NBUF):
            start_gather(b)

        def body(g, _):
            off = base + g * group
            for b in range(NBUF):
                wait_gather(b)
                start_store(off, b)
            pltpu.sync_copy(idx_hbm.at[pl.ds(off + group, group)], idx_v)
            for b in range(NBUF):
                wait_store(off, b)
                start_gather(b)
            return ()

        lax.fori_loop(0, n_groups - 1, body, (), unroll=False)

        off = base + (n_groups - 1) * group
        for b in range(NBUF):
            wait_gather(b)
            start_store(off, b)
        for b in range(NBUF):
            wait_store(off, b)

    return lookup


def kernel(p_sequences, table):
    B, L = p_sequences.shape
    V, D = table.shape
    idx_flat = p_sequences.reshape(B * L)
    lookup = _make_lookup(B * L, D)
    out = lookup(idx_flat, table)
    return out.reshape(B, L, D)
